# baseline (speedup 1.0000x reference)
import functools

import jax
import jax.numpy as jnp
from jax import lax
from jax.experimental import pallas as pl
from jax.experimental.pallas import tpu as pltpu
from jax.experimental.pallas import tpu_sc as plsc

N_NODES = 10000
D_IN = 128
D_HID = 64
N_GRAPHS = 64
ROW_BLK = 1000
GRID_M = N_NODES // ROW_BLK


def _mm_body(x_ref, w_ref, aa_ref, h_ref, asad_ref):
    h = jnp.dot(x_ref[...], w_ref[...], preferred_element_type=jnp.float32)
    h_ref[...] = h
    asad_ref[...] = jnp.dot(h, aa_ref[...], preferred_element_type=jnp.float32)


def _input_proj(x, W, a_src, a_dst):
    aa = jnp.stack([a_src, a_dst], axis=1)
    d_in = x.shape[1]
    return pl.pallas_call(
        _mm_body,
        grid=(GRID_M,),
        in_specs=[
            pl.BlockSpec((ROW_BLK, d_in), lambda i: (i, 0)),
            pl.BlockSpec((d_in, D_HID), lambda i: (0, 0)),
            pl.BlockSpec((D_HID, 2), lambda i: (0, 0)),
        ],
        out_specs=[
            pl.BlockSpec((ROW_BLK, D_HID), lambda i: (i, 0)),
            pl.BlockSpec((ROW_BLK, 2), lambda i: (i, 0)),
        ],
        out_shape=[
            jax.ShapeDtypeStruct((N_NODES, D_HID), jnp.float32),
            jax.ShapeDtypeStruct((N_NODES, 2), jnp.float32),
        ],
    )(x, W, aa)


def _merge1_body(n0_ref, n1_ref, dall_ref, h_ref, asad_ref, b_ref,
                 w2_ref, aa2_ref, h2_ref, asad2_ref):
    asad = asad_ref[...]
    e = asad[:, 0] + asad[:, 1]
    wself = jnp.exp(jnp.where(e < 0, 0.2 * e, e))
    den = jnp.sum(dall_ref[...], axis=1) + wself + 1e-16
    num = n0_ref[...] + n1_ref[...] + wself[:, None] * h_ref[...]
    o = num / den[:, None] + b_ref[...]
    o = jnp.where(o > 0, o, jnp.exp(o) - 1.0)
    h2 = jnp.dot(o, w2_ref[...], preferred_element_type=jnp.float32)
    h2_ref[...] = h2
    asad2_ref[...] = jnp.dot(h2, aa2_ref[...], preferred_element_type=jnp.float32)


def _merge_layer1(n0, n1, dall, h1, asad1, b1, W2, a_src2, a_dst2):
    aa2 = jnp.stack([a_src2, a_dst2], axis=1)
    return pl.pallas_call(
        _merge1_body,
        grid=(GRID_M,),
        in_specs=[
            pl.BlockSpec((ROW_BLK, D_HID), lambda i: (i, 0)),
            pl.BlockSpec((ROW_BLK, D_HID), lambda i: (i, 0)),
            pl.BlockSpec((ROW_BLK, NW), lambda i: (i, 0)),
            pl.BlockSpec((ROW_BLK, D_HID), lambda i: (i, 0)),
            pl.BlockSpec((ROW_BLK, 2), lambda i: (i, 0)),
            pl.BlockSpec((1, D_HID), lambda i: (0, 0)),
            pl.BlockSpec((D_HID, D_HID), lambda i: (0, 0)),
            pl.BlockSpec((D_HID, 2), lambda i: (0, 0)),
        ],
        out_specs=[
            pl.BlockSpec((ROW_BLK, D_HID), lambda i: (i, 0)),
            pl.BlockSpec((ROW_BLK, 2), lambda i: (i, 0)),
        ],
        out_shape=[
            jax.ShapeDtypeStruct((N_NODES, D_HID), jnp.float32),
            jax.ShapeDtypeStruct((N_NODES, 2), jnp.float32),
        ],
    )(n0, n1, dall, h1, asad1, b1.reshape(1, D_HID), W2, aa2)


def _merge2_body(n0_ref, n1_ref, dall_ref, h_ref, asad_ref, b_ref,
                 batch_ref, wc_ref, bc_ref, out_ref, gacc_ref, cacc_ref):
    i = pl.program_id(0)
    asad = asad_ref[...]
    e = asad[:, 0] + asad[:, 1]
    wself = jnp.exp(jnp.where(e < 0, 0.2 * e, e))
    den = jnp.sum(dall_ref[...], axis=1) + wself + 1e-16
    num = n0_ref[...] + n1_ref[...] + wself[:, None] * h_ref[...]
    o = num / den[:, None] + b_ref[...]

    gid = lax.broadcasted_iota(jnp.int32, (ROW_BLK, N_GRAPHS), 1)
    onehot = (batch_ref[...] == gid).astype(jnp.float32)
    g_part = lax.dot_general(onehot, o, (((0,), (0,)), ((), ())),
                             preferred_element_type=jnp.float32)
    c_part = jnp.sum(onehot, axis=0)[:, None]

    @pl.when(i == 0)
    def _init():
        gacc_ref[...] = jnp.zeros_like(gacc_ref)
        cacc_ref[...] = jnp.zeros_like(cacc_ref)

    gacc_ref[...] += g_part
    cacc_ref[...] += c_part

    @pl.when(i == GRID_M - 1)
    def _final():
        cnt = jnp.maximum(cacc_ref[...], 1.0)
        g = gacc_ref[...] / cnt
        logits = jnp.dot(g, wc_ref[...], preferred_element_type=jnp.float32) + bc_ref[...]
        m = jnp.max(logits, axis=1, keepdims=True)
        lse = m + jnp.log(jnp.sum(jnp.exp(logits - m), axis=1, keepdims=True))
        out_ref[...] = logits - lse


def _merge_layer2(n0, n1, dall, h2, asad2, b2, batch, Wc, bc):
    return pl.pallas_call(
        _merge2_body,
        grid=(GRID_M,),
        in_specs=[
            pl.BlockSpec((ROW_BLK, D_HID), lambda i: (i, 0)),
            pl.BlockSpec((ROW_BLK, D_HID), lambda i: (i, 0)),
            pl.BlockSpec((ROW_BLK, NW), lambda i: (i, 0)),
            pl.BlockSpec((ROW_BLK, D_HID), lambda i: (i, 0)),
            pl.BlockSpec((ROW_BLK, 2), lambda i: (i, 0)),
            pl.BlockSpec((1, D_HID), lambda i: (0, 0)),
            pl.BlockSpec((ROW_BLK, 1), lambda i: (i, 0)),
            pl.BlockSpec((D_HID, 2), lambda i: (0, 0)),
            pl.BlockSpec((1, 2), lambda i: (0, 0)),
        ],
        out_specs=[
            pl.BlockSpec((N_GRAPHS, 2), lambda i: (0, 0)),
            pl.BlockSpec((N_GRAPHS, D_HID), lambda i: (0, 0)),
            pl.BlockSpec((N_GRAPHS, 1), lambda i: (0, 0)),
        ],
        out_shape=[
            jax.ShapeDtypeStruct((N_GRAPHS, 2), jnp.float32),
            jax.ShapeDtypeStruct((N_GRAPHS, D_HID), jnp.float32),
            jax.ShapeDtypeStruct((N_GRAPHS, 1), jnp.float32),
        ],
    )(n0, n1, dall, h2, asad2, b2.reshape(1, D_HID), batch.reshape(N_NODES, 1),
      Wc, bc.reshape(1, 2))


N_EDGES_K = 320000
NUM_CORES = 2
NUM_SUBCORES = 16
NW = NUM_CORES * NUM_SUBCORES
CHUNK = 128
NCHUNK = 80
EPT = NCHUNK * CHUNK
E_PAD = EPT * NW
N_PAD = 10240
ROWS_PT = N_PAD // NUM_SUBCORES


def _edge_body(src_hbm, dst_hbm, as_hbm, ad_hbm, h_hbm, numer_out, denom_out,
               as_v, ad_v, srcL, dstL, wv, denom_v, rows0, rows1,
               numer_sh, sem0, sem1, semS0, semS1):
    cid = lax.axis_index("c")
    sid = lax.axis_index("s")
    wid = sid * NUM_CORES + cid

    pltpu.sync_copy(as_hbm, as_v)
    pltpu.sync_copy(ad_hbm, ad_v)
    pltpu.sync_copy(src_hbm.at[pl.ds(wid * NCHUNK, NCHUNK)], srcL)
    pltpu.sync_copy(dst_hbm.at[pl.ds(wid * NCHUNK, NCHUNK)], dstL)

    def _zero_den(j, _):
        denom_v[pl.ds(j * 16, 16)] = jnp.zeros((16,), jnp.float32)
        return 0
    lax.fori_loop(0, N_PAD // 16, _zero_den, 0)

    def _zero_rows(j, _):
        for q in range(4):
            rows0[j, pl.ds(q * 16, 16)] = jnp.zeros((16,), jnp.float32)
        return 0
    lax.fori_loop(0, CHUNK, _zero_rows, 0)
    for k in range(ROWS_PT // CHUNK):
        off = sid * ROWS_PT + k * CHUNK
        pltpu.sync_copy(rows0, numer_sh.at[pl.ds(off, CHUNK)])
    plsc.subcore_barrier()

    def _compute_w(c):
        for j8 in range(8):
            sidx = srcL[c, pl.ds(j8 * 16, 16)]
            didx = dstL[c, pl.ds(j8 * 16, 16)]
            e = plsc.load_gather(as_v, [sidx]) + plsc.load_gather(ad_v, [didx])
            e = jnp.where(e < 0, 0.2 * e, e)
            w = jnp.exp(e)
            wv[pl.ds(j8 * 16, 16)] = w
            plsc.addupdate_scatter(denom_v, [didx], w)

    def _scale(rows):
        for j in range(CHUNK):
            jv = jnp.full((16,), j, jnp.int32)
            wj = plsc.load_gather(wv, [jv])
            for q in range(4):
                rows[j, pl.ds(q * 16, 16)] = rows[j, pl.ds(q * 16, 16)] * wj

    pltpu.async_copy(h_hbm.at[srcL.at[0]], rows0, sem0)

    def _pair(k, _):
        c0 = 2 * k
        c1 = 2 * k + 1
        _compute_w(c0)
        pltpu.make_async_copy(h_hbm.at[srcL.at[c0]], rows0, sem0).wait()
        _scale(rows0)

        @pl.when(k > 0)
        def _drain_prev():
            pltpu.make_async_copy(rows1, numer_sh.at[dstL.at[c1]], semS1).wait()

        pltpu.async_copy(h_hbm.at[srcL.at[c1]], rows1, sem1)
        sc0 = pltpu.async_copy(rows0, numer_sh.at[dstL.at[c0]], semS0, add=True)
        _compute_w(c1)
        pltpu.make_async_copy(h_hbm.at[srcL.at[c1]], rows1, sem1).wait()
        _scale(rows1)
        sc0.wait()

        @pl.when(k < NCHUNK // 2 - 1)
        def _prefetch():
            pltpu.async_copy(h_hbm.at[srcL.at[c0 + 2]], rows0, sem0)

        pltpu.async_copy(rows1, numer_sh.at[dstL.at[c1]], semS1, add=True)
        return 0
    lax.fori_loop(0, NCHUNK // 2, _pair, 0)
    pltpu.make_async_copy(rows1, numer_sh.at[dstL.at[NCHUNK - 1]], semS1).wait()

    pltpu.sync_copy(denom_v, denom_out.at[cid, sid])
    plsc.subcore_barrier()
    base = sid * ROWS_PT
    pltpu.sync_copy(numer_sh.at[pl.ds(base, ROWS_PT)],
                    numer_out.at[cid, pl.ds(base, ROWS_PT)])


@functools.lru_cache(maxsize=1)
def _make_edge_kernel():
  mesh = plsc.VectorSubcoreMesh(core_axis_name="c", subcore_axis_name="s",
                                num_cores=NUM_CORES,
                                num_subcores=NUM_SUBCORES)
  return pl.kernel(
    _edge_body,
    out_type=[
        jax.ShapeDtypeStruct((NUM_CORES, N_PAD, D_HID), jnp.float32),
        jax.ShapeDtypeStruct((NUM_CORES, NUM_SUBCORES, N_PAD), jnp.float32),
    ],
    mesh=mesh,
    compiler_params=pltpu.CompilerParams(needs_layout_passes=False,
                                         use_tc_tiling_on_sc=False),
    scratch_types=[
        pltpu.VMEM((N_PAD,), jnp.float32),
        pltpu.VMEM((N_PAD,), jnp.float32),
        pltpu.VMEM((NCHUNK, CHUNK), jnp.int32),
        pltpu.VMEM((NCHUNK, CHUNK), jnp.int32),
        pltpu.VMEM((CHUNK,), jnp.float32),
        pltpu.VMEM((N_PAD,), jnp.float32),
        pltpu.VMEM((CHUNK, D_HID), jnp.float32),
        pltpu.VMEM((CHUNK, D_HID), jnp.float32),
        pltpu.VMEM_SHARED((N_PAD, D_HID), jnp.float32),
        pltpu.SemaphoreType.DMA,
        pltpu.SemaphoreType.DMA,
        pltpu.SemaphoreType.DMA,
        pltpu.SemaphoreType.DMA,
    ],
  )


def _edge_phase(src2d, dst2d, asad, h):
    as_arr = jnp.pad(asad[:, 0], (0, N_PAD - N_NODES))
    ad_arr = jnp.pad(asad[:, 1], (0, N_PAD - N_NODES))
    numer, denom = _make_edge_kernel()(src2d, dst2d, as_arr, ad_arr, h)
    return numer[0], numer[1], denom.reshape(NW, N_PAD).T


def kernel(x, edge_index, batch, W1, a_src1, a_dst1, b1, W2, a_src2, a_dst2, b2,
           Wc, bc):
    src2d = jnp.concatenate(
        [edge_index[0], jnp.zeros((E_PAD - N_EDGES_K,), jnp.int32)]
    ).reshape(NW * NCHUNK, CHUNK)
    dst2d = jnp.concatenate(
        [edge_index[1], jnp.full((E_PAD - N_EDGES_K,), N_NODES, jnp.int32)]
    ).reshape(NW * NCHUNK, CHUNK)
    h1, asad1 = _input_proj(x, W1, a_src1, a_dst1)
    n0, n1, dall = _edge_phase(src2d, dst2d, asad1, h1)
    h2, asad2 = _merge_layer1(n0, n1, dall, h1, asad1, b1, W2, a_src2, a_dst2)
    n0, n1, dall = _edge_phase(src2d, dst2d, asad2, h2)
    out, _, _ = _merge_layer2(n0, n1, dall, h2, asad2, b2, batch, Wc, bc)
    return out


# speedup vs baseline: 1.2750x; 1.2750x over previous
import functools

import jax
import jax.numpy as jnp
from jax import lax
from jax.experimental import pallas as pl
from jax.experimental.pallas import tpu as pltpu
from jax.experimental.pallas import tpu_sc as plsc

N_NODES = 10000
D_IN = 128
D_HID = 64
N_GRAPHS = 64
ROW_BLK = 1000
GRID_M = N_NODES // ROW_BLK


def _mm_body(x_ref, w_ref, aa_ref, h_ref, asad_ref):
    h = jnp.dot(x_ref[...], w_ref[...], preferred_element_type=jnp.float32)
    h_ref[...] = h
    asad_ref[...] = jnp.dot(h, aa_ref[...], preferred_element_type=jnp.float32)


def _input_proj(x, W, a_src, a_dst):
    aa = jnp.stack([a_src, a_dst], axis=1)
    d_in = x.shape[1]
    return pl.pallas_call(
        _mm_body,
        grid=(GRID_M,),
        in_specs=[
            pl.BlockSpec((ROW_BLK, d_in), lambda i: (i, 0)),
            pl.BlockSpec((d_in, D_HID), lambda i: (0, 0)),
            pl.BlockSpec((D_HID, 2), lambda i: (0, 0)),
        ],
        out_specs=[
            pl.BlockSpec((ROW_BLK, D_HID), lambda i: (i, 0)),
            pl.BlockSpec((ROW_BLK, 2), lambda i: (i, 0)),
        ],
        out_shape=[
            jax.ShapeDtypeStruct((N_NODES, D_HID), jnp.float32),
            jax.ShapeDtypeStruct((N_NODES, 2), jnp.float32),
        ],
    )(x, W, aa)


def _merge1_body(n0_ref, n1_ref, dall_ref, h_ref, asad_ref, b_ref,
                 w2_ref, aa2_ref, h2_ref, asad2_ref):
    asad = asad_ref[...]
    e = asad[:, 0] + asad[:, 1]
    wself = jnp.exp(jnp.where(e < 0, 0.2 * e, e))
    den = jnp.sum(dall_ref[...], axis=1) + wself + 1e-16
    num = n0_ref[...] + n1_ref[...] + wself[:, None] * h_ref[...]
    o = num / den[:, None] + b_ref[...]
    o = jnp.where(o > 0, o, jnp.exp(o) - 1.0)
    h2 = jnp.dot(o, w2_ref[...], preferred_element_type=jnp.float32)
    h2_ref[...] = h2
    asad2_ref[...] = jnp.dot(h2, aa2_ref[...], preferred_element_type=jnp.float32)


def _merge_layer1(n0, n1, dall, h1, asad1, b1, W2, a_src2, a_dst2):
    aa2 = jnp.stack([a_src2, a_dst2], axis=1)
    return pl.pallas_call(
        _merge1_body,
        grid=(GRID_M,),
        in_specs=[
            pl.BlockSpec((ROW_BLK, D_HID), lambda i: (i, 0)),
            pl.BlockSpec((ROW_BLK, D_HID), lambda i: (i, 0)),
            pl.BlockSpec((ROW_BLK, NW), lambda i: (i, 0)),
            pl.BlockSpec((ROW_BLK, D_HID), lambda i: (i, 0)),
            pl.BlockSpec((ROW_BLK, 2), lambda i: (i, 0)),
            pl.BlockSpec((1, D_HID), lambda i: (0, 0)),
            pl.BlockSpec((D_HID, D_HID), lambda i: (0, 0)),
            pl.BlockSpec((D_HID, 2), lambda i: (0, 0)),
        ],
        out_specs=[
            pl.BlockSpec((ROW_BLK, D_HID), lambda i: (i, 0)),
            pl.BlockSpec((ROW_BLK, 2), lambda i: (i, 0)),
        ],
        out_shape=[
            jax.ShapeDtypeStruct((N_NODES, D_HID), jnp.float32),
            jax.ShapeDtypeStruct((N_NODES, 2), jnp.float32),
        ],
    )(n0, n1, dall, h1, asad1, b1.reshape(1, D_HID), W2, aa2)


def _merge2_body(n0_ref, n1_ref, dall_ref, h_ref, asad_ref, b_ref,
                 batch_ref, wc_ref, bc_ref, out_ref, gacc_ref, cacc_ref):
    i = pl.program_id(0)
    asad = asad_ref[...]
    e = asad[:, 0] + asad[:, 1]
    wself = jnp.exp(jnp.where(e < 0, 0.2 * e, e))
    den = jnp.sum(dall_ref[...], axis=1) + wself + 1e-16
    num = n0_ref[...] + n1_ref[...] + wself[:, None] * h_ref[...]
    o = num / den[:, None] + b_ref[...]

    gid = lax.broadcasted_iota(jnp.int32, (ROW_BLK, N_GRAPHS), 1)
    onehot = (batch_ref[...] == gid).astype(jnp.float32)
    g_part = lax.dot_general(onehot, o, (((0,), (0,)), ((), ())),
                             preferred_element_type=jnp.float32)
    c_part = jnp.sum(onehot, axis=0)[:, None]

    @pl.when(i == 0)
    def _init():
        gacc_ref[...] = jnp.zeros_like(gacc_ref)
        cacc_ref[...] = jnp.zeros_like(cacc_ref)

    gacc_ref[...] += g_part
    cacc_ref[...] += c_part

    @pl.when(i == GRID_M - 1)
    def _final():
        cnt = jnp.maximum(cacc_ref[...], 1.0)
        g = gacc_ref[...] / cnt
        logits = jnp.dot(g, wc_ref[...], preferred_element_type=jnp.float32) + bc_ref[...]
        m = jnp.max(logits, axis=1, keepdims=True)
        lse = m + jnp.log(jnp.sum(jnp.exp(logits - m), axis=1, keepdims=True))
        out_ref[...] = logits - lse


def _merge_layer2(n0, n1, dall, h2, asad2, b2, batch, Wc, bc):
    return pl.pallas_call(
        _merge2_body,
        grid=(GRID_M,),
        in_specs=[
            pl.BlockSpec((ROW_BLK, D_HID), lambda i: (i, 0)),
            pl.BlockSpec((ROW_BLK, D_HID), lambda i: (i, 0)),
            pl.BlockSpec((ROW_BLK, NW), lambda i: (i, 0)),
            pl.BlockSpec((ROW_BLK, D_HID), lambda i: (i, 0)),
            pl.BlockSpec((ROW_BLK, 2), lambda i: (i, 0)),
            pl.BlockSpec((1, D_HID), lambda i: (0, 0)),
            pl.BlockSpec((ROW_BLK, 1), lambda i: (i, 0)),
            pl.BlockSpec((D_HID, 2), lambda i: (0, 0)),
            pl.BlockSpec((1, 2), lambda i: (0, 0)),
        ],
        out_specs=[
            pl.BlockSpec((N_GRAPHS, 2), lambda i: (0, 0)),
            pl.BlockSpec((N_GRAPHS, D_HID), lambda i: (0, 0)),
            pl.BlockSpec((N_GRAPHS, 1), lambda i: (0, 0)),
        ],
        out_shape=[
            jax.ShapeDtypeStruct((N_GRAPHS, 2), jnp.float32),
            jax.ShapeDtypeStruct((N_GRAPHS, D_HID), jnp.float32),
            jax.ShapeDtypeStruct((N_GRAPHS, 1), jnp.float32),
        ],
    )(n0, n1, dall, h2, asad2, b2.reshape(1, D_HID), batch.reshape(N_NODES, 1),
      Wc, bc.reshape(1, 2))


N_EDGES_K = 320000
NUM_CORES = 2
NUM_SUBCORES = 16
NW = NUM_CORES * NUM_SUBCORES
CHUNK = 128
NCHUNK = 80
EPT = NCHUNK * CHUNK
E_PAD = EPT * NW
N_PAD = 10240
ROWS_PT = N_PAD // NUM_SUBCORES


def _edge_body(src_hbm, dst_hbm, as_hbm, ad_hbm, h_hbm, numer_out, denom_out,
               as_v, ad_v, srcL, dstL, wv, denom_v, rows0, rows1,
               numer_sh, sem0, sem1, semS0, semS1):
    cid = lax.axis_index("c")
    sid = lax.axis_index("s")
    wid = sid * NUM_CORES + cid

    pltpu.sync_copy(as_hbm, as_v)
    pltpu.sync_copy(ad_hbm, ad_v)
    pltpu.sync_copy(src_hbm.at[pl.ds(wid * NCHUNK, NCHUNK)], srcL)
    pltpu.sync_copy(dst_hbm.at[pl.ds(wid * NCHUNK, NCHUNK)], dstL)

    def _zero_den(j, _):
        denom_v[pl.ds(j * 16, 16)] = jnp.zeros((16,), jnp.float32)
        return 0
    lax.fori_loop(0, N_PAD // 16, _zero_den, 0)

    def _zero_rows(j, _):
        for q in range(4):
            rows0[j, pl.ds(q * 16, 16)] = jnp.zeros((16,), jnp.float32)
        return 0
    lax.fori_loop(0, CHUNK, _zero_rows, 0)
    for k in range(ROWS_PT // CHUNK):
        off = sid * ROWS_PT + k * CHUNK
        pltpu.sync_copy(rows0, numer_sh.at[pl.ds(off, CHUNK)])
    plsc.subcore_barrier()

    def _compute_w(c):
        for j8 in range(8):
            sidx = srcL[c, pl.ds(j8 * 16, 16)]
            didx = dstL[c, pl.ds(j8 * 16, 16)]
            e = plsc.load_gather(as_v, [sidx]) + plsc.load_gather(ad_v, [didx])
            e = jnp.where(e < 0, 0.2 * e, e)
            w = jnp.exp(e)
            wv[pl.ds(j8 * 16, 16)] = w
            plsc.addupdate_scatter(denom_v, [didx], w)

    def _scale_scatter(c, rows):
        for j in range(CHUNK):
            jv = jnp.full((16,), j, jnp.int32)
            wj = plsc.load_gather(wv, [jv])
            for q in range(4):
                rows[j, pl.ds(q * 16, 16)] = rows[j, pl.ds(q * 16, 16)] * wj
        pltpu.sync_copy(rows, numer_sh.at[dstL.at[c]], add=True)

    pltpu.async_copy(h_hbm.at[srcL.at[0]], rows0, sem0)

    def _pair(k, _):
        c0 = 2 * k
        c1 = 2 * k + 1
        _compute_w(c0)
        pltpu.make_async_copy(h_hbm.at[srcL.at[c0]], rows0, sem0).wait()
        pltpu.async_copy(h_hbm.at[srcL.at[c1]], rows1, sem1)
        _scale_scatter(c0, rows0)
        _compute_w(c1)
        pltpu.make_async_copy(h_hbm.at[srcL.at[c1]], rows1, sem1).wait()

        @pl.when(k < NCHUNK // 2 - 1)
        def _prefetch():
            pltpu.async_copy(h_hbm.at[srcL.at[c0 + 2]], rows0, sem0)

        _scale_scatter(c1, rows1)
        return 0
    lax.fori_loop(0, NCHUNK // 2, _pair, 0)

    pltpu.sync_copy(denom_v, denom_out.at[cid, sid])
    plsc.subcore_barrier()
    base = sid * ROWS_PT
    pltpu.sync_copy(numer_sh.at[pl.ds(base, ROWS_PT)],
                    numer_out.at[cid, pl.ds(base, ROWS_PT)])


@functools.lru_cache(maxsize=1)
def _make_edge_kernel():
  mesh = plsc.VectorSubcoreMesh(core_axis_name="c", subcore_axis_name="s",
                                num_cores=NUM_CORES,
                                num_subcores=NUM_SUBCORES)
  return pl.kernel(
    _edge_body,
    out_type=[
        jax.ShapeDtypeStruct((NUM_CORES, N_PAD, D_HID), jnp.float32),
        jax.ShapeDtypeStruct((NUM_CORES, NUM_SUBCORES, N_PAD), jnp.float32),
    ],
    mesh=mesh,
    compiler_params=pltpu.CompilerParams(needs_layout_passes=False,
                                         use_tc_tiling_on_sc=False),
    scratch_types=[
        pltpu.VMEM((N_PAD,), jnp.float32),
        pltpu.VMEM((N_PAD,), jnp.float32),
        pltpu.VMEM((NCHUNK, CHUNK), jnp.int32),
        pltpu.VMEM((NCHUNK, CHUNK), jnp.int32),
        pltpu.VMEM((CHUNK,), jnp.float32),
        pltpu.VMEM((N_PAD,), jnp.float32),
        pltpu.VMEM((CHUNK, D_HID), jnp.float32),
        pltpu.VMEM((CHUNK, D_HID), jnp.float32),
        pltpu.VMEM_SHARED((N_PAD, D_HID), jnp.float32),
        pltpu.SemaphoreType.DMA,
        pltpu.SemaphoreType.DMA,
        pltpu.SemaphoreType.DMA,
        pltpu.SemaphoreType.DMA,
    ],
  )


def _edge_phase(src2d, dst2d, asad, h):
    as_arr = jnp.pad(asad[:, 0], (0, N_PAD - N_NODES))
    ad_arr = jnp.pad(asad[:, 1], (0, N_PAD - N_NODES))
    numer, denom = _make_edge_kernel()(src2d, dst2d, as_arr, ad_arr, h)
    return numer[0], numer[1], denom.reshape(NW, N_PAD).T


def kernel(x, edge_index, batch, W1, a_src1, a_dst1, b1, W2, a_src2, a_dst2, b2,
           Wc, bc):
    src2d = jnp.concatenate(
        [edge_index[0], jnp.zeros((E_PAD - N_EDGES_K,), jnp.int32)]
    ).reshape(NW * NCHUNK, CHUNK)
    dst2d = jnp.concatenate(
        [edge_index[1], jnp.full((E_PAD - N_EDGES_K,), N_NODES, jnp.int32)]
    ).reshape(NW * NCHUNK, CHUNK)
    h1, asad1 = _input_proj(x, W1, a_src1, a_dst1)
    n0, n1, dall = _edge_phase(src2d, dst2d, asad1, h1)
    h2, asad2 = _merge_layer1(n0, n1, dall, h1, asad1, b1, W2, a_src2, a_dst2)
    n0, n1, dall = _edge_phase(src2d, dst2d, asad2, h2)
    out, _, _ = _merge_layer2(n0, n1, dall, h2, asad2, b2, batch, Wc, bc)
    return out
